# baseline (speedup 1.0000x reference)
import functools

import jax
import jax.numpy as jnp
from jax import lax
from jax.experimental import pallas as pl
from jax.experimental.pallas import tpu as pltpu
from jax.experimental.pallas import tpu_sc as plsc

T = 2048
D = 1024
E = 8
K = 2
DFF = 512
DFF_S = 1024

BT = 256
NB = (T * K) // BT + E - 1
P_X = NB * BT
P_Y = (NB + 1) * BT
BTD = 256
NBD = T // BTD

NC = 2
NS = 16
NW = NC * NS
A_BPW = (T * K) // NW
CH = 128

D2 = D // 2

_NT = (((1,), (1,)), ((), ()))


def _pack_bf16(v):
    vb = v.astype(jnp.bfloat16)
    n = vb.shape[1] // 2
    lo = lax.bitcast_convert_type(vb[:, :n], jnp.int16).astype(jnp.int32) & 0xFFFF
    hi = lax.bitcast_convert_type(vb[:, n:], jnp.int16).astype(jnp.int32)
    return (hi << 16) | lo


def _unpack_bf16(p):
    lo = lax.bitcast_convert_type(p.astype(jnp.int16), jnp.bfloat16)
    hi = lax.bitcast_convert_type(
        lax.shift_right_logical(p, 16).astype(jnp.int16), jnp.bfloat16)
    return jnp.concatenate([lo, hi], axis=1)


def _router_body(x_ref, wg_ref, pos_ref, w_ref, xb16_ref, meta_ref):
    x = x_ref[...]
    xb16_ref[...] = _pack_bf16(x)
    wg = wg_ref[...]
    logits = lax.dot_general(x, wg, _NT, preferred_element_type=jnp.float32)
    m = jnp.max(logits, axis=1, keepdims=True)
    ex = jnp.exp(logits - m)
    scores = ex / jnp.sum(ex, axis=1, keepdims=True)
    iota = lax.broadcasted_iota(jnp.int32, (T, E), 1)
    m1 = jnp.max(scores, axis=1, keepdims=True)
    i1 = jnp.min(jnp.where(scores == m1, iota, E), axis=1, keepdims=True)
    s2 = jnp.where(iota == i1, -jnp.inf, scores)
    m2 = jnp.max(s2, axis=1, keepdims=True)
    i2 = jnp.min(jnp.where(s2 == m2, iota, E), axis=1, keepdims=True)
    den = m1 + m2
    w_ref[...] = jnp.concatenate([m1 / den, m2 / den], axis=1)

    oh1 = (iota == i1).astype(jnp.float32)
    oh2 = (iota == i2).astype(jnp.float32)
    ohs = oh1 + oh2
    cb = 128
    ri = lax.broadcasted_iota(jnp.int32, (cb, cb), 0)
    ci = lax.broadcasted_iota(jnp.int32, (cb, cb), 1)
    tri = (ci < ri).astype(jnp.float32)
    parts = []
    run = jnp.zeros((1, E), jnp.float32)
    for gi in range(T // cb):
        xg = ohs[gi * cb:(gi + 1) * cb]
        sg = lax.dot_general(tri, xg, (((1,), (0,)), ((), ())),
                             preferred_element_type=jnp.float32) + run
        run = run + jnp.sum(xg, axis=0, keepdims=True)
        parts.append(sg)
    s = jnp.concatenate(parts, axis=0)
    counts = run
    nb = jnp.floor((counts + (BT - 1)) * (1.0 / BT))
    ei = lax.broadcasted_iota(jnp.int32, (E, E), 0)
    ej = lax.broadcasted_iota(jnp.int32, (E, E), 1)
    tri_e = (ei < ej).astype(jnp.float32)
    blk_start = lax.dot_general(nb, tri_e, (((1,), (0,)), ((), ())),
                                preferred_element_type=jnp.float32)
    pad_off = blk_start * BT
    num_real = jnp.sum(nb, axis=1, keepdims=True)
    tgt = pad_off + s
    pos1 = jnp.sum(oh1 * tgt, axis=1, keepdims=True)
    pos2 = jnp.sum(oh2 * tgt, axis=1, keepdims=True)
    pos_ref[...] = jnp.concatenate([pos1, pos2], axis=1).astype(jnp.int32)

    bi = lax.broadcasted_iota(jnp.int32, (NB, 1), 0)
    num_real_i = num_real.astype(jnp.int32)
    bc = jnp.minimum(bi, num_real_i - 1)
    bs_b = blk_start.astype(jnp.int32)
    wblk = jnp.sum((bc >= bs_b).astype(jnp.int32), axis=1, keepdims=True) - 1
    valid = (bi < num_real_i).astype(jnp.int32)
    oblk = jnp.where(valid == 1, bi, NB)
    meta_ref[...] = jnp.concatenate([wblk, bc, oblk, valid], axis=1)


def _router_call(x, Wg):
    return pl.pallas_call(
        _router_body,
        out_shape=(
            jax.ShapeDtypeStruct((T, K), jnp.int32),
            jax.ShapeDtypeStruct((T, K), jnp.float32),
            jax.ShapeDtypeStruct((T, D2), jnp.int32),
            jax.ShapeDtypeStruct((NB, 4), jnp.int32),
        ),
    )(x, Wg)


@functools.cache
def _sc_bucket_x():
    mesh = plsc.VectorSubcoreMesh(
        core_axis_name="c", subcore_axis_name="s", num_cores=NC)

    @functools.partial(
        pl.kernel,
        mesh=mesh,
        out_type=jax.ShapeDtypeStruct((P_X, D2), jnp.int32),
        scratch_types=[
            pltpu.VMEM((CH,), jnp.int32),
            pltpu.VMEM((CH,), jnp.int32),
            pltpu.VMEM((CH, D2), jnp.int32),
            pltpu.SemaphoreType.DMA,
            pltpu.SemaphoreType.DMA,
        ],
    )
    def body(x_hbm, tok_hbm, pos_hbm, xs_hbm, tok_v, pos_v, rows_v, g_sem, s_sem):
        wid = lax.axis_index("s") * NC + lax.axis_index("c")
        base = wid * A_BPW
        for c in range(A_BPW // CH):
            off = base + c * CH
            pltpu.sync_copy(tok_hbm.at[pl.ds(off, CH)], tok_v)
            pltpu.sync_copy(pos_hbm.at[pl.ds(off, CH)], pos_v)
            pltpu.async_copy(x_hbm.at[tok_v], rows_v, g_sem).wait()
            pltpu.async_copy(rows_v, xs_hbm.at[pos_v], s_sem).wait()

    return body


@functools.cache
def _sc_gather_y():
    mesh = plsc.VectorSubcoreMesh(
        core_axis_name="c", subcore_axis_name="s", num_cores=NC)

    @functools.partial(
        pl.kernel,
        mesh=mesh,
        out_type=(
            jax.ShapeDtypeStruct((T, D2), jnp.int32),
            jax.ShapeDtypeStruct((T, D2), jnp.int32),
        ),
        scratch_types=[
            pltpu.VMEM((CH,), jnp.int32),
            pltpu.VMEM((CH, D2), jnp.int32),
            pltpu.SemaphoreType.DMA,
        ],
    )
    def body(yp_hbm, pos_hbm, y1_hbm, y2_hbm, idx_v, rows_v, g_sem):
        wid = lax.axis_index("s") * NC + lax.axis_index("c")
        base = wid * A_BPW
        for c in range(A_BPW // CH):
            off = base + c * CH
            pltpu.sync_copy(pos_hbm.at[pl.ds(off, CH)], idx_v)
            pltpu.async_copy(yp_hbm.at[idx_v], rows_v, g_sem).wait()

            @pl.when(wid < NW // 2)
            def _():
                pltpu.sync_copy(rows_v, y1_hbm.at[pl.ds(off, CH)])

            @pl.when(wid >= NW // 2)
            def _():
                pltpu.sync_copy(rows_v, y2_hbm.at[pl.ds(off - T, CH)])

    return body


def _ffn_body(meta_ref, xs_ref, wg_ref, wu_ref, wd_ref, out_ref):
    g = pl.program_id(0)

    @pl.when(meta_ref[g, 3] == 1)
    def _():
        xb = _unpack_bf16(xs_ref[...])
        wg = wg_ref[0].astype(jnp.bfloat16)
        wu = wu_ref[0].astype(jnp.bfloat16)
        gg = lax.dot_general(xb, wg, _NT, preferred_element_type=jnp.float32)
        uu = lax.dot_general(xb, wu, _NT, preferred_element_type=jnp.float32)
        hh = (gg * jax.nn.sigmoid(gg) * uu).astype(jnp.bfloat16)
        wd = wd_ref[0].astype(jnp.bfloat16)
        out_ref[...] = _pack_bf16(
            lax.dot_general(hh, wd, _NT, preferred_element_type=jnp.float32))


def _ffn_call(meta, xs, w_gate, w_up, w_down):
    grid_spec = pltpu.PrefetchScalarGridSpec(
        num_scalar_prefetch=1,
        grid=(NB,),
        in_specs=[
            pl.BlockSpec((BT, D2), lambda g, m: (m[g, 1], 0)),
            pl.BlockSpec((1, DFF, D), lambda g, m: (m[g, 0], 0, 0)),
            pl.BlockSpec((1, DFF, D), lambda g, m: (m[g, 0], 0, 0)),
            pl.BlockSpec((1, D, DFF), lambda g, m: (m[g, 0], 0, 0)),
        ],
        out_specs=pl.BlockSpec((BT, D2), lambda g, m: (m[g, 2], 0)),
    )
    return pl.pallas_call(
        _ffn_body,
        grid_spec=grid_spec,
        out_shape=jax.ShapeDtypeStruct((P_Y, D2), jnp.int32),
    )(meta, xs, w_gate, w_up, w_down)


def _shared_body(x_ref, wsg_ref, wsu_ref, wsd_ref, out_ref):
    xb = _unpack_bf16(x_ref[...])
    wsg = wsg_ref[...].astype(jnp.bfloat16)
    wsu = wsu_ref[...].astype(jnp.bfloat16)
    sg = lax.dot_general(xb, wsg, _NT, preferred_element_type=jnp.float32)
    su = lax.dot_general(xb, wsu, _NT, preferred_element_type=jnp.float32)
    sh = (sg * jax.nn.sigmoid(sg) * su).astype(jnp.bfloat16)
    wsd = wsd_ref[...].astype(jnp.bfloat16)
    out_ref[...] = _pack_bf16(
        lax.dot_general(sh, wsd, _NT, preferred_element_type=jnp.float32))


def _shared_call(x32, ws_gate, ws_up, ws_down):
    return pl.pallas_call(
        _shared_body,
        grid=(NBD,),
        in_specs=[
            pl.BlockSpec((BTD, D2), lambda i: (i, 0)),
            pl.BlockSpec((DFF_S, D), lambda i: (0, 0)),
            pl.BlockSpec((DFF_S, D), lambda i: (0, 0)),
            pl.BlockSpec((D, DFF_S), lambda i: (0, 0)),
        ],
        out_specs=pl.BlockSpec((BTD, D2), lambda i: (i, 0)),
        out_shape=jax.ShapeDtypeStruct((T, D2), jnp.int32),
    )(x32, ws_gate, ws_up, ws_down)


def _combine_body(sh_ref, y1_ref, y2_ref, w1_ref, w2_ref, out_ref):
    sd = _unpack_bf16(sh_ref[...]).astype(jnp.float32)
    out_ref[...] = (sd + w1_ref[...] * _unpack_bf16(y1_ref[...]).astype(jnp.float32)
                    + w2_ref[...] * _unpack_bf16(y2_ref[...]).astype(jnp.float32))


def _combine_call(shared, y1, y2, w1, w2):
    return pl.pallas_call(
        _combine_body,
        grid=(NBD,),
        in_specs=[
            pl.BlockSpec((BTD, D2), lambda i: (i, 0)),
            pl.BlockSpec((BTD, D2), lambda i: (i, 0)),
            pl.BlockSpec((BTD, D2), lambda i: (i, 0)),
            pl.BlockSpec((BTD, 1), lambda i: (i, 0)),
            pl.BlockSpec((BTD, 1), lambda i: (i, 0)),
        ],
        out_specs=pl.BlockSpec((BTD, D), lambda i: (i, 0)),
        out_shape=jax.ShapeDtypeStruct((T, D), jnp.float32),
    )(shared, y1, y2, w1, w2)


def kernel(hidden_states, Wg, w_gate, w_up, w_down, ws_gate, ws_up, ws_down):
    x = hidden_states
    pos, w, x32, meta = _router_call(x, Wg)
    pos_flat = jnp.concatenate([pos[:, 0], pos[:, 1]])
    tok = jnp.concatenate([jnp.arange(T, dtype=jnp.int32)] * K)
    xs = _sc_bucket_x()(x32, tok, pos_flat)
    shared = _shared_call(x32, ws_gate, ws_up, ws_down)
    y_padded = _ffn_call(meta, xs, w_gate, w_up, w_down)
    y1, y2 = _sc_gather_y()(y_padded, pos_flat)
    return _combine_call(shared, y1, y2, w[:, 0:1], w[:, 1:2])


# speedup vs baseline: 1.0459x; 1.0459x over previous
import functools

import jax
import jax.numpy as jnp
from jax import lax
from jax.experimental import pallas as pl
from jax.experimental.pallas import tpu as pltpu
from jax.experimental.pallas import tpu_sc as plsc

T = 2048
D = 1024
E = 8
K = 2
DFF = 512
DFF_S = 1024

BT = 256
NB = (T * K) // BT + E - 1
P_X = NB * BT
P_Y = (NB + 1) * BT
BTD = 256
NBD = T // BTD

NC = 2
NS = 16
NW = NC * NS
A_BPW = (T * K) // NW
CH = 128

D2 = D // 2

_NT = (((1,), (1,)), ((), ()))


def _pack_bf16(v):
    vb = v.astype(jnp.bfloat16)
    n = vb.shape[1] // 2
    lo = lax.bitcast_convert_type(vb[:, :n], jnp.int16).astype(jnp.int32) & 0xFFFF
    hi = lax.bitcast_convert_type(vb[:, n:], jnp.int16).astype(jnp.int32)
    return (hi << 16) | lo


def _unpack_bf16(p):
    lo = lax.bitcast_convert_type(p.astype(jnp.int16), jnp.bfloat16)
    hi = lax.bitcast_convert_type(
        lax.shift_right_logical(p, 16).astype(jnp.int16), jnp.bfloat16)
    return jnp.concatenate([lo, hi], axis=1)


def _router_body(x_ref, wg_ref, pos_ref, w1_ref, w2_ref, xb16_ref, meta_ref):
    x = x_ref[...]
    xb16_ref[...] = _pack_bf16(x)
    wg = wg_ref[...]
    logits = lax.dot_general(x, wg, _NT, preferred_element_type=jnp.float32)
    m = jnp.max(logits, axis=1, keepdims=True)
    ex = jnp.exp(logits - m)
    scores = ex / jnp.sum(ex, axis=1, keepdims=True)
    iota = lax.broadcasted_iota(jnp.int32, (T, E), 1)
    m1 = jnp.max(scores, axis=1, keepdims=True)
    i1 = jnp.min(jnp.where(scores == m1, iota, E), axis=1, keepdims=True)
    s2 = jnp.where(iota == i1, -jnp.inf, scores)
    m2 = jnp.max(s2, axis=1, keepdims=True)
    i2 = jnp.min(jnp.where(s2 == m2, iota, E), axis=1, keepdims=True)
    den = m1 + m2
    w1_ref[...] = m1 / den
    w2_ref[...] = m2 / den

    oh1 = (iota == i1).astype(jnp.float32)
    oh2 = (iota == i2).astype(jnp.float32)
    ohs = oh1 + oh2
    cb = 128
    ri = lax.broadcasted_iota(jnp.int32, (cb, cb), 0)
    ci = lax.broadcasted_iota(jnp.int32, (cb, cb), 1)
    tri = (ci < ri).astype(jnp.float32)
    parts = []
    run = jnp.zeros((1, E), jnp.float32)
    for gi in range(T // cb):
        xg = ohs[gi * cb:(gi + 1) * cb]
        sg = lax.dot_general(tri, xg, (((1,), (0,)), ((), ())),
                             preferred_element_type=jnp.float32) + run
        run = run + jnp.sum(xg, axis=0, keepdims=True)
        parts.append(sg)
    s = jnp.concatenate(parts, axis=0)
    counts = run
    nb = jnp.floor((counts + (BT - 1)) * (1.0 / BT))
    ei = lax.broadcasted_iota(jnp.int32, (E, E), 0)
    ej = lax.broadcasted_iota(jnp.int32, (E, E), 1)
    tri_e = (ei < ej).astype(jnp.float32)
    blk_start = lax.dot_general(nb, tri_e, (((1,), (0,)), ((), ())),
                                preferred_element_type=jnp.float32)
    pad_off = blk_start * BT
    num_real = jnp.sum(nb, axis=1, keepdims=True)
    tgt = pad_off + s
    pos1 = jnp.sum(oh1 * tgt, axis=1, keepdims=True)
    pos2 = jnp.sum(oh2 * tgt, axis=1, keepdims=True)
    pos_ref[...] = jnp.concatenate([pos1, pos2], axis=1).astype(jnp.int32)

    bi = lax.broadcasted_iota(jnp.int32, (NB, 1), 0)
    num_real_i = num_real.astype(jnp.int32)
    bc = jnp.minimum(bi, num_real_i - 1)
    bs_b = blk_start.astype(jnp.int32)
    wblk = jnp.sum((bc >= bs_b).astype(jnp.int32), axis=1, keepdims=True) - 1
    valid = (bi < num_real_i).astype(jnp.int32)
    oblk = jnp.where(valid == 1, bi, NB)
    meta_ref[...] = jnp.concatenate([wblk, bc, oblk, valid], axis=1)


def _router_call(x, Wg):
    return pl.pallas_call(
        _router_body,
        out_shape=(
            jax.ShapeDtypeStruct((T, K), jnp.int32),
            jax.ShapeDtypeStruct((T, 1), jnp.float32),
            jax.ShapeDtypeStruct((T, 1), jnp.float32),
            jax.ShapeDtypeStruct((T, D2), jnp.int32),
            jax.ShapeDtypeStruct((NB, 4), jnp.int32),
        ),
    )(x, Wg)


@functools.cache
def _sc_bucket_x():
    mesh = plsc.VectorSubcoreMesh(
        core_axis_name="c", subcore_axis_name="s", num_cores=NC)

    @functools.partial(
        pl.kernel,
        mesh=mesh,
        out_type=jax.ShapeDtypeStruct((P_X, D2), jnp.int32),
        scratch_types=[
            pltpu.VMEM((CH,), jnp.int32),
            pltpu.VMEM((CH,), jnp.int32),
            pltpu.VMEM((CH, D2), jnp.int32),
            pltpu.SemaphoreType.DMA,
            pltpu.SemaphoreType.DMA,
        ],
    )
    def body(x_hbm, tok_hbm, pos_hbm, xs_hbm, tok_v, pos_v, rows_v, g_sem, s_sem):
        wid = lax.axis_index("s") * NC + lax.axis_index("c")
        base = wid * A_BPW
        for c in range(A_BPW // CH):
            off = base + c * CH
            pltpu.sync_copy(tok_hbm.at[pl.ds(off, CH)], tok_v)
            pltpu.sync_copy(pos_hbm.at[pl.ds(off, CH)], pos_v)
            pltpu.async_copy(x_hbm.at[tok_v], rows_v, g_sem).wait()
            pltpu.async_copy(rows_v, xs_hbm.at[pos_v], s_sem).wait()

    return body


@functools.cache
def _sc_gather_y():
    mesh = plsc.VectorSubcoreMesh(
        core_axis_name="c", subcore_axis_name="s", num_cores=NC)

    @functools.partial(
        pl.kernel,
        mesh=mesh,
        out_type=(
            jax.ShapeDtypeStruct((T, D2), jnp.int32),
            jax.ShapeDtypeStruct((T, D2), jnp.int32),
        ),
        scratch_types=[
            pltpu.VMEM((CH,), jnp.int32),
            pltpu.VMEM((CH, D2), jnp.int32),
            pltpu.SemaphoreType.DMA,
        ],
    )
    def body(yp_hbm, pos_hbm, y1_hbm, y2_hbm, idx_v, rows_v, g_sem):
        wid = lax.axis_index("s") * NC + lax.axis_index("c")
        base = wid * A_BPW
        for c in range(A_BPW // CH):
            off = base + c * CH
            pltpu.sync_copy(pos_hbm.at[pl.ds(off, CH)], idx_v)
            pltpu.async_copy(yp_hbm.at[idx_v], rows_v, g_sem).wait()

            @pl.when(wid < NW // 2)
            def _():
                pltpu.sync_copy(rows_v, y1_hbm.at[pl.ds(off, CH)])

            @pl.when(wid >= NW // 2)
            def _():
                pltpu.sync_copy(rows_v, y2_hbm.at[pl.ds(off - T, CH)])

    return body


def _ffn_body(meta_ref, xs_ref, wg_ref, wu_ref, wd_ref, out_ref):
    g = pl.program_id(0)

    @pl.when(meta_ref[g, 3] == 1)
    def _():
        xb = _unpack_bf16(xs_ref[...])
        wg = wg_ref[0].astype(jnp.bfloat16)
        wu = wu_ref[0].astype(jnp.bfloat16)
        gg = lax.dot_general(xb, wg, _NT, preferred_element_type=jnp.float32)
        uu = lax.dot_general(xb, wu, _NT, preferred_element_type=jnp.float32)
        hh = (gg * jax.nn.sigmoid(gg) * uu).astype(jnp.bfloat16)
        wd = wd_ref[0].astype(jnp.bfloat16)
        out_ref[...] = _pack_bf16(
            lax.dot_general(hh, wd, _NT, preferred_element_type=jnp.float32))


def _ffn_call(meta, xs, w_gate, w_up, w_down):
    grid_spec = pltpu.PrefetchScalarGridSpec(
        num_scalar_prefetch=1,
        grid=(NB,),
        in_specs=[
            pl.BlockSpec((BT, D2), lambda g, m: (m[g, 1], 0)),
            pl.BlockSpec((1, DFF, D), lambda g, m: (m[g, 0], 0, 0)),
            pl.BlockSpec((1, DFF, D), lambda g, m: (m[g, 0], 0, 0)),
            pl.BlockSpec((1, D, DFF), lambda g, m: (m[g, 0], 0, 0)),
        ],
        out_specs=pl.BlockSpec((BT, D2), lambda g, m: (m[g, 2], 0)),
    )
    return pl.pallas_call(
        _ffn_body,
        grid_spec=grid_spec,
        out_shape=jax.ShapeDtypeStruct((P_Y, D2), jnp.int32),
    )(meta, xs, w_gate, w_up, w_down)


def _combine_body(x_ref, wsg_ref, wsu_ref, wsd_ref, y1_ref, y2_ref,
                  w1_ref, w2_ref, out_ref):
    xb = _unpack_bf16(x_ref[...])
    wsg = wsg_ref[...].astype(jnp.bfloat16)
    wsu = wsu_ref[...].astype(jnp.bfloat16)
    sg = lax.dot_general(xb, wsg, _NT, preferred_element_type=jnp.float32)
    su = lax.dot_general(xb, wsu, _NT, preferred_element_type=jnp.float32)
    sh = (sg * jax.nn.sigmoid(sg) * su).astype(jnp.bfloat16)
    wsd = wsd_ref[...].astype(jnp.bfloat16)
    sd = lax.dot_general(sh, wsd, _NT, preferred_element_type=jnp.float32)
    out_ref[...] = (sd + w1_ref[...] * _unpack_bf16(y1_ref[...]).astype(jnp.float32)
                    + w2_ref[...] * _unpack_bf16(y2_ref[...]).astype(jnp.float32))


def _combine_call(x32, ws_gate, ws_up, ws_down, y1, y2, w1, w2):
    return pl.pallas_call(
        _combine_body,
        grid=(NBD,),
        in_specs=[
            pl.BlockSpec((BTD, D2), lambda i: (i, 0)),
            pl.BlockSpec((DFF_S, D), lambda i: (0, 0)),
            pl.BlockSpec((DFF_S, D), lambda i: (0, 0)),
            pl.BlockSpec((D, DFF_S), lambda i: (0, 0)),
            pl.BlockSpec((BTD, D2), lambda i: (i, 0)),
            pl.BlockSpec((BTD, D2), lambda i: (i, 0)),
            pl.BlockSpec((BTD, 1), lambda i: (i, 0)),
            pl.BlockSpec((BTD, 1), lambda i: (i, 0)),
        ],
        out_specs=pl.BlockSpec((BTD, D), lambda i: (i, 0)),
        out_shape=jax.ShapeDtypeStruct((T, D), jnp.float32),
    )(x32, ws_gate, ws_up, ws_down, y1, y2, w1, w2)


def kernel(hidden_states, Wg, w_gate, w_up, w_down, ws_gate, ws_up, ws_down):
    x = hidden_states
    pos, w1, w2, x32, meta = _router_call(x, Wg)
    pos_flat = jnp.concatenate([pos[:, 0], pos[:, 1]])
    tok = jnp.concatenate([jnp.arange(T, dtype=jnp.int32)] * K)
    xs = _sc_bucket_x()(x32, tok, pos_flat)
    y_padded = _ffn_call(meta, xs, w_gate, w_up, w_down)
    y1, y2 = _sc_gather_y()(y_padded, pos_flat)
    return _combine_call(x32, ws_gate, ws_up, ws_down, y1, y2, w1, w2)


# speedup vs baseline: 1.1514x; 1.1009x over previous
import functools

import jax
import jax.numpy as jnp
from jax import lax
from jax.experimental import pallas as pl
from jax.experimental.pallas import tpu as pltpu
from jax.experimental.pallas import tpu_sc as plsc

T = 2048
D = 1024
E = 8
K = 2
DFF = 512
DFF_S = 1024

BT = 512
NB = (T * K) // BT + E - 1
P_X = NB * BT
P_Y = (NB + 1) * BT
BTD = 256
NBD = T // BTD

NC = 2
NS = 16
NW = NC * NS
A_BPW = (T * K) // NW
CH = 128

D2 = D // 2

_NT = (((1,), (1,)), ((), ()))


def _pack_bf16(v):
    vb = v.astype(jnp.bfloat16)
    n = vb.shape[1] // 2
    lo = lax.bitcast_convert_type(vb[:, :n], jnp.int16).astype(jnp.int32) & 0xFFFF
    hi = lax.bitcast_convert_type(vb[:, n:], jnp.int16).astype(jnp.int32)
    return (hi << 16) | lo


def _unpack_bf16(p):
    lo = lax.bitcast_convert_type(p.astype(jnp.int16), jnp.bfloat16)
    hi = lax.bitcast_convert_type(
        lax.shift_right_logical(p, 16).astype(jnp.int16), jnp.bfloat16)
    return jnp.concatenate([lo, hi], axis=1)


def _router_body(x_ref, wg_ref, pos_ref, w1_ref, w2_ref, xb16_ref, meta_ref):
    x = x_ref[...]
    xb16_ref[...] = _pack_bf16(x)
    wg = wg_ref[...]
    logits = lax.dot_general(x, wg, _NT, preferred_element_type=jnp.float32)
    m = jnp.max(logits, axis=1, keepdims=True)
    ex = jnp.exp(logits - m)
    scores = ex / jnp.sum(ex, axis=1, keepdims=True)
    iota = lax.broadcasted_iota(jnp.int32, (T, E), 1)
    m1 = jnp.max(scores, axis=1, keepdims=True)
    i1 = jnp.min(jnp.where(scores == m1, iota, E), axis=1, keepdims=True)
    s2 = jnp.where(iota == i1, -jnp.inf, scores)
    m2 = jnp.max(s2, axis=1, keepdims=True)
    i2 = jnp.min(jnp.where(s2 == m2, iota, E), axis=1, keepdims=True)
    den = m1 + m2
    w1_ref[...] = m1 / den
    w2_ref[...] = m2 / den

    oh1 = (iota == i1).astype(jnp.float32)
    oh2 = (iota == i2).astype(jnp.float32)
    ohs = oh1 + oh2
    cb = 128
    ri = lax.broadcasted_iota(jnp.int32, (cb, cb), 0)
    ci = lax.broadcasted_iota(jnp.int32, (cb, cb), 1)
    tri = (ci < ri).astype(jnp.float32)
    parts = []
    run = jnp.zeros((1, E), jnp.float32)
    for gi in range(T // cb):
        xg = ohs[gi * cb:(gi + 1) * cb]
        sg = lax.dot_general(tri, xg, (((1,), (0,)), ((), ())),
                             preferred_element_type=jnp.float32) + run
        run = run + jnp.sum(xg, axis=0, keepdims=True)
        parts.append(sg)
    s = jnp.concatenate(parts, axis=0)
    counts = run
    nb = jnp.floor((counts + (BT - 1)) * (1.0 / BT))
    ei = lax.broadcasted_iota(jnp.int32, (E, E), 0)
    ej = lax.broadcasted_iota(jnp.int32, (E, E), 1)
    tri_e = (ei < ej).astype(jnp.float32)
    blk_start = lax.dot_general(nb, tri_e, (((1,), (0,)), ((), ())),
                                preferred_element_type=jnp.float32)
    pad_off = blk_start * BT
    num_real = jnp.sum(nb, axis=1, keepdims=True)
    tgt = pad_off + s
    pos1 = jnp.sum(oh1 * tgt, axis=1, keepdims=True)
    pos2 = jnp.sum(oh2 * tgt, axis=1, keepdims=True)
    pos_ref[...] = jnp.concatenate([pos1, pos2], axis=1).astype(jnp.int32)

    bi = lax.broadcasted_iota(jnp.int32, (NB, 1), 0)
    num_real_i = num_real.astype(jnp.int32)
    bc = jnp.minimum(bi, num_real_i - 1)
    bs_b = blk_start.astype(jnp.int32)
    wblk = jnp.sum((bc >= bs_b).astype(jnp.int32), axis=1, keepdims=True) - 1
    valid = (bi < num_real_i).astype(jnp.int32)
    oblk = jnp.where(valid == 1, bi, NB)
    meta_ref[...] = jnp.concatenate([wblk, bc, oblk, valid], axis=1)


def _router_call(x, Wg):
    return pl.pallas_call(
        _router_body,
        out_shape=(
            jax.ShapeDtypeStruct((T, K), jnp.int32),
            jax.ShapeDtypeStruct((T, 1), jnp.float32),
            jax.ShapeDtypeStruct((T, 1), jnp.float32),
            jax.ShapeDtypeStruct((T, D2), jnp.int32),
            jax.ShapeDtypeStruct((NB, 4), jnp.int32),
        ),
    )(x, Wg)


@functools.cache
def _sc_bucket_x():
    mesh = plsc.VectorSubcoreMesh(
        core_axis_name="c", subcore_axis_name="s", num_cores=NC)

    @functools.partial(
        pl.kernel,
        mesh=mesh,
        out_type=jax.ShapeDtypeStruct((P_X, D2), jnp.int32),
        scratch_types=[
            pltpu.VMEM((CH,), jnp.int32),
            pltpu.VMEM((CH,), jnp.int32),
            pltpu.VMEM((CH, D2), jnp.int32),
            pltpu.SemaphoreType.DMA,
            pltpu.SemaphoreType.DMA,
        ],
    )
    def body(x_hbm, tok_hbm, pos_hbm, xs_hbm, tok_v, pos_v, rows_v, g_sem, s_sem):
        wid = lax.axis_index("s") * NC + lax.axis_index("c")
        base = wid * A_BPW
        for c in range(A_BPW // CH):
            off = base + c * CH
            pltpu.sync_copy(tok_hbm.at[pl.ds(off, CH)], tok_v)
            pltpu.sync_copy(pos_hbm.at[pl.ds(off, CH)], pos_v)
            pltpu.async_copy(x_hbm.at[tok_v], rows_v, g_sem).wait()
            pltpu.async_copy(rows_v, xs_hbm.at[pos_v], s_sem).wait()

    return body


@functools.cache
def _sc_gather_y():
    mesh = plsc.VectorSubcoreMesh(
        core_axis_name="c", subcore_axis_name="s", num_cores=NC)

    @functools.partial(
        pl.kernel,
        mesh=mesh,
        out_type=(
            jax.ShapeDtypeStruct((T, D2), jnp.int32),
            jax.ShapeDtypeStruct((T, D2), jnp.int32),
        ),
        scratch_types=[
            pltpu.VMEM((CH,), jnp.int32),
            pltpu.VMEM((CH, D2), jnp.int32),
            pltpu.SemaphoreType.DMA,
        ],
    )
    def body(yp_hbm, pos_hbm, y1_hbm, y2_hbm, idx_v, rows_v, g_sem):
        wid = lax.axis_index("s") * NC + lax.axis_index("c")
        base = wid * A_BPW
        for c in range(A_BPW // CH):
            off = base + c * CH
            pltpu.sync_copy(pos_hbm.at[pl.ds(off, CH)], idx_v)
            pltpu.async_copy(yp_hbm.at[idx_v], rows_v, g_sem).wait()

            @pl.when(wid < NW // 2)
            def _():
                pltpu.sync_copy(rows_v, y1_hbm.at[pl.ds(off, CH)])

            @pl.when(wid >= NW // 2)
            def _():
                pltpu.sync_copy(rows_v, y2_hbm.at[pl.ds(off - T, CH)])

    return body


def _ffn_body(meta_ref, xs_ref, wg_ref, wu_ref, wd_ref, out_ref):
    g = pl.program_id(0)

    @pl.when(meta_ref[g, 3] == 1)
    def _():
        xb = _unpack_bf16(xs_ref[...])
        wg = wg_ref[0].astype(jnp.bfloat16)
        wu = wu_ref[0].astype(jnp.bfloat16)
        gg = lax.dot_general(xb, wg, _NT, preferred_element_type=jnp.float32)
        uu = lax.dot_general(xb, wu, _NT, preferred_element_type=jnp.float32)
        hh = (gg * jax.nn.sigmoid(gg) * uu).astype(jnp.bfloat16)
        wd = wd_ref[0].astype(jnp.bfloat16)
        out_ref[...] = _pack_bf16(
            lax.dot_general(hh, wd, _NT, preferred_element_type=jnp.float32))


def _ffn_call(meta, xs, w_gate, w_up, w_down):
    grid_spec = pltpu.PrefetchScalarGridSpec(
        num_scalar_prefetch=1,
        grid=(NB,),
        in_specs=[
            pl.BlockSpec((BT, D2), lambda g, m: (m[g, 1], 0)),
            pl.BlockSpec((1, DFF, D), lambda g, m: (m[g, 0], 0, 0)),
            pl.BlockSpec((1, DFF, D), lambda g, m: (m[g, 0], 0, 0)),
            pl.BlockSpec((1, D, DFF), lambda g, m: (m[g, 0], 0, 0)),
        ],
        out_specs=pl.BlockSpec((BT, D2), lambda g, m: (m[g, 2], 0)),
    )
    return pl.pallas_call(
        _ffn_body,
        grid_spec=grid_spec,
        out_shape=jax.ShapeDtypeStruct((P_Y, D2), jnp.int32),
    )(meta, xs, w_gate, w_up, w_down)


def _combine_body(x_ref, wsg_ref, wsu_ref, wsd_ref, y1_ref, y2_ref,
                  w1_ref, w2_ref, out_ref):
    xb = _unpack_bf16(x_ref[...])
    wsg = wsg_ref[...].astype(jnp.bfloat16)
    wsu = wsu_ref[...].astype(jnp.bfloat16)
    sg = lax.dot_general(xb, wsg, _NT, preferred_element_type=jnp.float32)
    su = lax.dot_general(xb, wsu, _NT, preferred_element_type=jnp.float32)
    sh = (sg * jax.nn.sigmoid(sg) * su).astype(jnp.bfloat16)
    wsd = wsd_ref[...].astype(jnp.bfloat16)
    sd = lax.dot_general(sh, wsd, _NT, preferred_element_type=jnp.float32)
    out_ref[...] = (sd + w1_ref[...] * _unpack_bf16(y1_ref[...]).astype(jnp.float32)
                    + w2_ref[...] * _unpack_bf16(y2_ref[...]).astype(jnp.float32))


def _combine_call(x32, ws_gate, ws_up, ws_down, y1, y2, w1, w2):
    return pl.pallas_call(
        _combine_body,
        grid=(NBD,),
        in_specs=[
            pl.BlockSpec((BTD, D2), lambda i: (i, 0)),
            pl.BlockSpec((DFF_S, D), lambda i: (0, 0)),
            pl.BlockSpec((DFF_S, D), lambda i: (0, 0)),
            pl.BlockSpec((D, DFF_S), lambda i: (0, 0)),
            pl.BlockSpec((BTD, D2), lambda i: (i, 0)),
            pl.BlockSpec((BTD, D2), lambda i: (i, 0)),
            pl.BlockSpec((BTD, 1), lambda i: (i, 0)),
            pl.BlockSpec((BTD, 1), lambda i: (i, 0)),
        ],
        out_specs=pl.BlockSpec((BTD, D), lambda i: (i, 0)),
        out_shape=jax.ShapeDtypeStruct((T, D), jnp.float32),
    )(x32, ws_gate, ws_up, ws_down, y1, y2, w1, w2)


def kernel(hidden_states, Wg, w_gate, w_up, w_down, ws_gate, ws_up, ws_down):
    x = hidden_states
    pos, w1, w2, x32, meta = _router_call(x, Wg)
    pos_flat = jnp.concatenate([pos[:, 0], pos[:, 1]])
    tok = jnp.concatenate([jnp.arange(T, dtype=jnp.int32)] * K)
    xs = _sc_bucket_x()(x32, tok, pos_flat)
    y_padded = _ffn_call(meta, xs, w_gate, w_up, w_down)
    y1, y2 = _sc_gather_y()(y_padded, pos_flat)
    return _combine_call(x32, ws_gate, ws_up, ws_down, y1, y2, w1, w2)
